# baseline (speedup 1.0000x reference)
import jax
import jax.numpy as jnp
from jax import lax
from jax.experimental import pallas as pl

BB = 8


def _body(x_ref, wp_ref, cp_ref, o0_ref, o1_ref):
    n = x_ref.shape[1]
    R = BB * n
    x = x_ref[...].reshape(R, 128)
    u = lax.dot_general(
        wp_ref[...], x,
        dimension_numbers=(((1,), (1,)), ((), ())),
        preferred_element_type=jnp.float32,
    )
    t = u.reshape(BB, n) + cp_ref[...]
    p1 = 1.0 / (1.0 + jnp.exp(-t))
    o1_ref[...] = p1
    o0_ref[...] = 1.0 - p1


def kernel(xs, W, b):
    B, N, D = xs.shape
    w = W[1] - W[0]
    c = b[1] - b[0]
    wp = w.reshape(1, D)
    cp = jnp.full((1, N), c, dtype=jnp.float32)
    p0, p1 = pl.pallas_call(
        _body,
        grid=(B // BB,),
        in_specs=[
            pl.BlockSpec((BB, N, D), lambda i: (i, 0, 0)),
            pl.BlockSpec((1, D), lambda i: (0, 0)),
            pl.BlockSpec((1, N), lambda i: (0, 0)),
        ],
        out_specs=[
            pl.BlockSpec((BB, N), lambda i: (i, 0)),
            pl.BlockSpec((BB, N), lambda i: (i, 0)),
        ],
        out_shape=[
            jax.ShapeDtypeStruct((B, N), jnp.float32),
            jax.ShapeDtypeStruct((B, N), jnp.float32),
        ],
    )(xs, wp, cp)
    return jnp.stack([p0, p1], axis=-1)
